# Initial kernel scaffold; baseline (speedup 1.0000x reference)
import jax
import jax.numpy as jnp
from jax.experimental import pallas as pl


def kernel(node_feat, edge_index, pos_enc, lin1_W, lin1_b, lin2_W, lin2_b, pe_lin_W, pe_lin_b, cor_W, cor_b, coef_W, coef_b, temp):
    raise NotImplementedError("write your pallas kernel here")


# speedup vs baseline: 5.9190x; 5.9190x over previous
import functools
from math import comb

import jax
import jax.numpy as jnp
from jax import lax
from jax.experimental import pallas as pl
from jax.experimental.pallas import tpu as pltpu
from jax.experimental.pallas import tpu_sc as plsc

NC, NS, LANES = 2, 16, 16
NW = NC * NS
CHUNK = 128
NPAD = 10240
ROWS_PER_TILE = NPAD // NS
BR = 512
NBLK = NPAD // BR
KORD = 4

f32 = jnp.float32


def _zero_shared(acc, zbuf, sid, sem):
    ncopies = ROWS_PER_TILE // CHUNK
    for k in range(ncopies):
        pltpu.async_copy(
            zbuf, acc.at[pl.ds(sid * ROWS_PER_TILE + k * CHUNK, CHUNK)], sem
        ).wait()


def _fill_buf(buf, value, width):
    vec = jnp.full((LANES,), value, dtype=f32)
    for r in range(CHUNK):
        for j in range(width // LANES):
            buf[r, pl.ds(j * LANES, LANES)] = vec


def _make_sc_degrees(nch):
    mesh = plsc.VectorSubcoreMesh(core_axis_name="c", subcore_axis_name="s", num_cores=NC, num_subcores=NS)

    def body(src4, dst4, out, acc1, acc2, ones_v, zbuf, sidx1, sidx2,
             sem0, sem1, sem2):
        cid = lax.axis_index("c")
        sid = lax.axis_index("s")
        wid = sid * NC + cid
        _fill_buf(ones_v, 1.0, LANES)
        _fill_buf(zbuf, 0.0, LANES)
        _zero_shared(acc1, zbuf, sid, sem0)
        _zero_shared(acc2, zbuf, sid, sem0)
        plsc.subcore_barrier()
        pltpu.sync_copy(src4.at[wid], sidx1)
        pltpu.sync_copy(dst4.at[wid], sidx2)
        for j in range(nch):
            pltpu.async_copy(ones_v, acc1.at[sidx1.at[j]], sem1, add=True).wait()
            pltpu.async_copy(ones_v, acc2.at[sidx2.at[j]], sem2, add=True).wait()
        plsc.subcore_barrier()
        rs = pl.ds(sid * ROWS_PER_TILE, ROWS_PER_TILE)
        pltpu.sync_copy(acc1.at[rs], out.at[cid, 0, rs])
        pltpu.sync_copy(acc2.at[rs], out.at[cid, 1, rs])

    return pl.kernel(
        body,
        out_type=jax.ShapeDtypeStruct((NC, 2, NPAD, LANES), f32),
        mesh=mesh,
        scratch_types=[
            pltpu.VMEM_SHARED((NPAD, LANES), f32),
            pltpu.VMEM_SHARED((NPAD, LANES), f32),
            pltpu.VMEM((CHUNK, LANES), f32),
            pltpu.VMEM((CHUNK, LANES), f32),
            pltpu.VMEM((nch, CHUNK), jnp.int32),
            pltpu.VMEM((nch, CHUNK), jnp.int32),
            pltpu.SemaphoreType.DMA,
            pltpu.SemaphoreType.DMA,
            pltpu.SemaphoreType.DMA,
        ],
    )


def _make_sc_prop(feat, nch, ew):
    mesh = plsc.VectorSubcoreMesh(core_axis_name="c", subcore_axis_name="s", num_cores=NC, num_subcores=NS)
    NBUF = 2 if feat >= 128 else 4

    def body(*refs):
        (z, src_flat, dst4, out, acc), rest = refs[:5], refs[5:]
        bufs = rest[:NBUF]
        gidx, sidx = rest[NBUF], rest[NBUF + 1]
        zsem = rest[NBUF + 2]
        gsems = rest[NBUF + 3:NBUF + 3 + NBUF]
        ssems = rest[NBUF + 3 + NBUF:NBUF + 3 + 2 * NBUF]
        cid = lax.axis_index("c")
        sid = lax.axis_index("s")
        wid = sid * NC + cid
        _fill_buf(bufs[0], 0.0, feat)
        _zero_shared(acc, bufs[0], sid, zsem)
        plsc.subcore_barrier()
        pltpu.sync_copy(src_flat.at[wid], gidx)
        pltpu.sync_copy(dst4.at[wid], sidx)
        gd = [None] * NBUF
        sd = [None] * NBUF
        for b in range(min(NBUF, nch)):
            gd[b] = pltpu.async_copy(
                z.at[gidx.at[pl.ds(b * CHUNK, CHUNK)]], bufs[b], gsems[b])
        for j in range(nch):
            b = j % NBUF
            gd[b].wait()
            sd[b] = pltpu.async_copy(bufs[b], acc.at[sidx.at[j]],
                                     ssems[b], add=True)
            nxt = j + NBUF
            if nxt < nch:
                sd[b].wait()
                gd[b] = pltpu.async_copy(
                    z.at[gidx.at[pl.ds(nxt * CHUNK, CHUNK)]], bufs[b], gsems[b])
        for b in range(min(NBUF, nch)):
            if sd[(nch - 1 - b) % NBUF] is not None:
                sd[(nch - 1 - b) % NBUF].wait()
                sd[(nch - 1 - b) % NBUF] = None
        plsc.subcore_barrier()
        rs = pl.ds(sid * ROWS_PER_TILE, ROWS_PER_TILE)
        pltpu.sync_copy(acc.at[rs], out.at[cid, rs])

    return pl.kernel(
        body,
        out_type=jax.ShapeDtypeStruct((NC, NPAD, feat), f32),
        mesh=mesh,
        scratch_types=[pltpu.VMEM_SHARED((NPAD, feat), f32)]
        + [pltpu.VMEM((CHUNK, feat), f32)] * NBUF
        + [pltpu.VMEM((ew,), jnp.int32), pltpu.VMEM((nch, CHUNK), jnp.int32)]
        + [pltpu.SemaphoreType.DMA] * (1 + 2 * NBUF),
    )


def _vspec(block, imap):
    return pl.BlockSpec(block, imap)


def _row_spec(width):
    return pl.BlockSpec((BR, width), lambda i: (i, 0))


def _full_spec(shape):
    nd = len(shape)
    return pl.BlockSpec(shape, lambda i: (0,) * nd)


def _tc_prep(node_feat_p, lin1_W, lin1_b, lin2_W, lin2_b,
             pos_enc_p, pe_lin_W, pe_lin_b, degs, rowmask):
    def body(x, w1, b1, w2, b2, pos, pw, pb, dg, rm,
             h_o, u0_o, pe0_o, v0_o, dis_o, dis2_o):
        h1 = jnp.maximum(jnp.dot(x[...], w1[...],
                                 preferred_element_type=f32) + b1[...], 0.0)
        h = jnp.dot(h1, w2[...], preferred_element_type=f32) + b2[...]
        deg = dg[0, 0] + dg[1, 0]
        deg2 = dg[0, 1] + dg[1, 1] + 1.0
        dis = jnp.where(deg > 0, lax.rsqrt(jnp.maximum(deg, 1e-30)), 0.0)
        dis2 = lax.rsqrt(deg2)
        pe0 = jnp.tanh(jnp.dot(pos[...], pw[...],
                               preferred_element_type=f32) + pb[...]) * rm[...]
        h_o[...] = h
        u0_o[...] = dis[:, :1] * h
        pe0_o[...] = pe0
        v0_o[...] = jnp.concatenate(
            [dis2 * pe0, jnp.zeros((BR, 128 - LANES), f32)], axis=1)
        dis_o[...] = dis
        dis2_o[...] = dis2

    out_shapes = [
        jax.ShapeDtypeStruct((NPAD, 128), f32),
        jax.ShapeDtypeStruct((NPAD, 128), f32),
        jax.ShapeDtypeStruct((NPAD, LANES), f32),
        jax.ShapeDtypeStruct((NPAD, 128), f32),
        jax.ShapeDtypeStruct((NPAD, LANES), f32),
        jax.ShapeDtypeStruct((NPAD, LANES), f32),
    ]
    return pl.pallas_call(
        body,
        grid=(NBLK,),
        in_specs=[
            _row_spec(128), _full_spec((128, 128)), _full_spec((1, 128)),
            _full_spec((128, 128)), _full_spec((1, 128)),
            _row_spec(LANES), _full_spec((LANES, LANES)),
            _full_spec((1, LANES)),
            pl.BlockSpec((2, 2, BR, LANES), lambda i: (0, 0, i, 0)),
            _row_spec(LANES),
        ],
        out_specs=[_row_spec(128), _row_spec(128), _row_spec(LANES),
                   _row_spec(128), _row_spec(LANES), _row_spec(LANES)],
        out_shape=out_shapes,
    )(node_feat_p, lin1_W, lin1_b.reshape(1, -1), lin2_W,
      lin2_b.reshape(1, -1), pos_enc_p, pe_lin_W, pe_lin_b.reshape(1, -1),
      degs, rowmask)


def _tc_step(p_m, s, dis16):
    def body(p, sp, d, pn_o, un_o):
        dis = d[:, :1]
        pn = p[...] - dis * (sp[0] + sp[1])
        pn_o[...] = pn
        un_o[...] = dis * pn

    return pl.pallas_call(
        body,
        grid=(NBLK,),
        in_specs=[_row_spec(128),
                  pl.BlockSpec((2, BR, 128), lambda i: (0, i, 0)),
                  _row_spec(LANES)],
        out_specs=[_row_spec(128), _row_spec(128)],
        out_shape=[jax.ShapeDtypeStruct((NPAD, 128), f32),
                   jax.ShapeDtypeStruct((NPAD, 128), f32)],
    )(p_m, s, dis16)


def _zcoefs(t):
    return [comb(KORD - t, j) * (2.0 ** (KORD - t - j)) * ((-1.0) ** j)
            for j in range(KORD - t + 1)]


def _tc_out0(ps, pe0, cw0, gparams0):
    zc = _zcoefs(0)
    c0 = comb(KORD, 0) / 2.0 ** KORD

    def body(p0, p1, p2, p3, p4, pe, cw, gp, out_o):
        z = (zc[0] * p0[...] + zc[1] * p1[...] + zc[2] * p2[...]
             + zc[3] * p3[...] + zc[4] * p4[...])
        glin = jnp.sum(pe[...] * cw[...], axis=1, keepdims=True) + gp[0]
        gamma = gp[1] * jax.nn.sigmoid(glin)
        out_o[...] = c0 * gamma * z

    return pl.pallas_call(
        body,
        grid=(NBLK,),
        in_specs=[_row_spec(128)] * 5 + [
            _row_spec(LANES), _full_spec((1, LANES)),
            pl.BlockSpec(memory_space=pltpu.SMEM)],
        out_specs=_row_spec(128),
        out_shape=jax.ShapeDtypeStruct((NPAD, 128), f32),
    )(*ps, pe0, cw0, gparams0)


def _tc_flash(t, pe_cur, raw_pe, tp, dis216, rowmask, cor_W, cor_b,
              cw_t, gparams_t, ps, out_acc, pe_alpha, pe_beta):
    zc = _zcoefs(t)
    ct = comb(KORD, t) / 2.0 ** KORD
    nps = len(ps)

    def body(*refs):
        (pe_full, pe_blk, raw, tpp, d2, rm, cw_c, cb_c, cwt, gp), rest = \
            refs[:10], refs[10:]
        p_refs = rest[:nps]
        oacc = rest[nps]
        out_o, pe_o, v_o = rest[nps + 1:]

        cor = cw_c[...]
        corb = cb_c[...]
        hc_blk = jnp.dot(pe_blk[...], cor, preferred_element_type=f32) + corb
        acc = jnp.zeros((BR, LANES), f32)
        for cb in range(NBLK):
            pe_c = pe_full[pl.ds(cb * BR, BR), :]
            hc_c = jnp.dot(pe_c, cor, preferred_element_type=f32) + corb
            s = lax.dot_general(hc_blk, hc_c, (((1,), (1,)), ((), ())),
                                preferred_element_type=f32)
            acc = acc + jnp.dot(jax.nn.sigmoid(s), pe_c,
                                preferred_element_type=f32)
        pe_corr = acc
        dis2 = d2[...]
        peb = pe_blk[...]
        pe_tpo = dis2 * (tpp[0, :, :LANES] + tpp[1, :, :LANES]) \
            + dis2 * dis2 * peb
        pe_new = (1.0 + pe_beta) * pe_tpo - pe_beta * pe_corr
        pe_new = pe_alpha * raw[...] + (1.0 - pe_alpha) * pe_new
        pe_new = jnp.tanh(pe_new) * rm[...]
        glin = jnp.sum(pe_new * cwt[...], axis=1, keepdims=True) + gp[0]
        gamma = gp[1] * jax.nn.sigmoid(glin)
        z = zc[0] * p_refs[0][...]
        for j in range(1, nps):
            z = z + zc[j] * p_refs[j][...]
        out_o[...] = oacc[...] + ct * gamma * z
        pe_o[...] = pe_new
        v_o[...] = jnp.concatenate(
            [dis2 * pe_new, jnp.zeros((BR, 128 - LANES), f32)], axis=1)

    return pl.pallas_call(
        body,
        grid=(NBLK,),
        in_specs=[
            _full_spec((NPAD, LANES)),
            _row_spec(LANES),
            _row_spec(LANES),
            pl.BlockSpec((2, BR, 128), lambda i: (0, i, 0)),
            _row_spec(LANES),
            _row_spec(LANES),
            _full_spec((LANES, LANES)),
            _full_spec((1, LANES)),
            _full_spec((1, LANES)),
            pl.BlockSpec(memory_space=pltpu.SMEM),
        ] + [_row_spec(128)] * nps + [_row_spec(128)],
        out_specs=[_row_spec(128), _row_spec(LANES), _row_spec(128)],
        out_shape=[jax.ShapeDtypeStruct((NPAD, 128), f32),
                   jax.ShapeDtypeStruct((NPAD, LANES), f32),
                   jax.ShapeDtypeStruct((NPAD, 128), f32)],
    )(pe_cur, pe_cur, raw_pe, tp, dis216, rowmask, cor_W,
      cor_b.reshape(1, -1), cw_t, gparams_t, *ps, out_acc)


def kernel(node_feat, edge_index, pos_enc, lin1_W, lin1_b, lin2_W, lin2_b,
           pe_lin_W, pe_lin_b, cor_W, cor_b, coef_W, coef_b, temp):
    n, in_dim = node_feat.shape
    e = edge_index.shape[1]
    pe_alpha, pe_beta = 0.1, 0.5

    ew = -(-e // (NW * CHUNK)) * CHUNK
    epad = ew * NW
    nch = ew // CHUNK

    trash = jnp.int32(n)
    row = edge_index[0]
    col = edge_index[1]
    pad = jnp.full((epad - e,), trash, dtype=jnp.int32)
    rowp = jnp.concatenate([row, pad])
    colp = jnp.concatenate([col, pad])
    src_flat = rowp.reshape(NW, ew)
    src4 = rowp.reshape(NW, nch, CHUNK)
    dst4 = colp.reshape(NW, nch, CHUNK)

    node_feat_p = jnp.zeros((NPAD, in_dim), f32).at[:n].set(node_feat)
    pos_enc_p = jnp.zeros((NPAD, pos_enc.shape[1]), f32).at[:n].set(pos_enc)
    rowmask = (jnp.arange(NPAD) < n).astype(f32)[:, None] * jnp.ones(
        (1, LANES), f32)

    TEMP = jax.nn.relu(temp)

    sc_degrees = _make_sc_degrees(nch)
    sc_prop128 = _make_sc_prop(128, nch, ew)

    degs = sc_degrees(src4, dst4)
    h, u, pe0, v, dis16, dis216 = _tc_prep(
        node_feat_p, lin1_W, lin1_b, lin2_W, lin2_b,
        pos_enc_p, pe_lin_W, pe_lin_b, degs, rowmask)

    ps = [h]
    for _ in range(KORD):
        s = sc_prop128(u, src_flat, dst4)
        p_next, u = _tc_step(ps[-1], s, dis16)
        ps.append(p_next)

    out = _tc_out0(ps, pe0, coef_W[0][None, :],
                   jnp.stack([coef_b[0], TEMP[0]]))

    pe_cur = pe0
    for t in range(1, KORD + 1):
        tp = sc_prop128(v, src_flat, dst4)
        out, pe_cur, v = _tc_flash(
            t, pe_cur, pe0, tp, dis216, rowmask, cor_W, cor_b,
            coef_W[t][None, :], jnp.stack([coef_b[t], TEMP[t]]),
            ps[t:], out, pe_alpha, pe_beta)

    return (out[:n], pe_cur[:n, :])
